# baseline (speedup 1.0000x reference)
import functools

import jax
import jax.numpy as jnp
from jax import lax
from jax.experimental import pallas as pl
from jax.experimental.pallas import tpu as pltpu, tpu_sc as plsc

B, N, D = 32, 4096, 64
NC, NS, L = 2, 16, 16
NW = NC * NS
NVEC = N // L
IDX_PER_DMA = 128
ROWS_CHUNK = 512
N_DMA = ROWS_CHUNK // IDX_PER_DMA
N_CHUNKS = N // ROWS_CHUNK


def _sc_body(x_hbm, w_hbm, ll_hbm, pi_hbm, inds_hbm,
             mean_hbm, xr_hbm, wr_hbm, llr_hbm, pir_hbm,
             inds_v, w_v, ll_v, pi_v,
             wr_v, llr_v, pir_v, wexp_v, rows_v, acc_v, sem):
  wid = lax.axis_index("s") * NC + lax.axis_index("c")
  xb_hbm = x_hbm.at[wid]
  xrb_hbm = xr_hbm.at[wid]

  pltpu.sync_copy(inds_hbm.at[wid], inds_v)
  pltpu.sync_copy(w_hbm.at[wid], w_v)
  pltpu.sync_copy(ll_hbm.at[wid], ll_v)
  pltpu.sync_copy(pi_hbm.at[wid], pi_v)

  def gather_body(j, mx):
    sl = pl.ds(j * L, L)
    idx16 = inds_v[sl]
    wr16 = plsc.load_gather(w_v, [idx16])
    wr_v[sl] = wr16
    llr_v[sl] = plsc.load_gather(ll_v, [idx16])
    pir_v[sl] = plsc.load_gather(pi_v, [idx16])
    return jnp.maximum(mx, wr16)

  mx16 = lax.fori_loop(0, NVEC, gather_body,
                       jnp.full((L,), -jnp.inf, jnp.float32))
  m = lax.reduce_max_p.bind(mx16, axes=(0,))

  def exp_body(j, s):
    sl = pl.ds(j * L, L)
    e16 = jnp.exp(wr_v[sl] - m)
    wexp_v[sl] = e16
    return s + e16

  s16 = lax.fori_loop(0, NVEC, exp_body, jnp.zeros((L,), jnp.float32))
  z = lax.reduce_sum_p.bind(s16, axes=(0,))

  pltpu.sync_copy(wr_v, wr_hbm.at[wid])
  pltpu.sync_copy(llr_v, llr_hbm.at[wid])
  pltpu.sync_copy(pir_v, pir_hbm.at[wid])

  def chunk_body(c, acc):
    cbase = c * ROWS_CHUNK
    copies = []
    for t in range(N_DMA):
      idx_ref = inds_v.at[pl.ds(cbase + t * IDX_PER_DMA, IDX_PER_DMA)]
      dst = rows_v.at[pl.ds(t * IDX_PER_DMA, IDX_PER_DMA)]
      copies.append(pltpu.async_copy(xb_hbm.at[idx_ref], dst, sem))
    for cp in copies:
      cp.wait()
    pltpu.sync_copy(rows_v, xrb_hbm.at[pl.ds(cbase, ROWS_CHUNK)])

    def row_body(i, acc):
      bvec = plsc.load_gather(wexp_v, [jnp.full((L,), cbase + i, jnp.int32)])
      return tuple(acc[k] + bvec * rows_v[i, pl.ds(k * L, L)]
                   for k in range(D // L))

    return lax.fori_loop(0, ROWS_CHUNK, row_body, acc)

  acc0 = tuple(jnp.zeros((L,), jnp.float32) for _ in range(D // L))
  acc = lax.fori_loop(0, N_CHUNKS, chunk_body, acc0)

  z_vec = jnp.full((L,), z, jnp.float32)
  for k in range(D // L):
    acc_v[pl.ds(k * L, L)] = acc[k] / z_vec
  pltpu.sync_copy(acc_v, mean_hbm.at[wid])


@jax.jit
def kernel(x, w, ll, prev_inds, inds):
  inds32 = inds.astype(jnp.int32)
  pi32 = prev_inds.astype(jnp.int32)

  mesh = plsc.VectorSubcoreMesh(core_axis_name="c", subcore_axis_name="s")
  run = pl.kernel(
      _sc_body,
      out_type=(
          jax.ShapeDtypeStruct((B, D), jnp.float32),
          jax.ShapeDtypeStruct((B, N, D), jnp.float32),
          jax.ShapeDtypeStruct((B, N), jnp.float32),
          jax.ShapeDtypeStruct((B, N), jnp.float32),
          jax.ShapeDtypeStruct((B, N), jnp.int32),
      ),
      mesh=mesh,
      compiler_params=pltpu.CompilerParams(needs_layout_passes=False,
                                           use_tc_tiling_on_sc=False),
      scratch_types=[
          pltpu.VMEM((N,), jnp.int32),
          pltpu.VMEM((N,), jnp.float32),
          pltpu.VMEM((N,), jnp.float32),
          pltpu.VMEM((N,), jnp.int32),
          pltpu.VMEM((N,), jnp.float32),
          pltpu.VMEM((N,), jnp.float32),
          pltpu.VMEM((N,), jnp.int32),
          pltpu.VMEM((N,), jnp.float32),
          pltpu.VMEM((ROWS_CHUNK, D), jnp.float32),
          pltpu.VMEM((D,), jnp.float32),
          pltpu.SemaphoreType.DMA,
      ],
  )
  mean, x_r, wr, llr, pir = run(x, w, ll, pi32, inds32)
  return (mean, x_r, wr, llr, pir.astype(prev_inds.dtype))


# speedup vs baseline: 1.3209x; 1.3209x over previous
import jax
import jax.numpy as jnp
from jax import lax
from jax.experimental import pallas as pl
from jax.experimental.pallas import tpu as pltpu, tpu_sc as plsc

B, N, D = 32, 4096, 64
NC, NS, L = 2, 16, 16
NVEC = N // L
RPG = 4
NG = D // RPG
NBUF = 2


def _sc_body(x_hbm, w_hbm, ll_hbm, pi_hbm, inds_hbm,
             mean_hbm, xr_hbm, wr_hbm, llr_hbm, pir_hbm,
             inds_v, w_v, ll_v, pi_v,
             wr_v, llr_v, pir_v, wexp_v, acc_v,
             ibuf0, ibuf1, obuf0, obuf1,
             sem_in0, sem_in1, sem_out0, sem_out1):
  wid = lax.axis_index("s") * NC + lax.axis_index("c")
  xb = x_hbm.at[wid]
  xrb = xr_hbm.at[wid]
  ibufs = (ibuf0, ibuf1)
  obufs = (obuf0, obuf1)
  sems_in = (sem_in0, sem_in1)
  sems_out = (sem_out0, sem_out1)

  in_dma = {}
  for g in range(NBUF):
    in_dma[g] = pltpu.async_copy(
        xb.at[pl.ds(g * RPG, RPG)], ibufs[g], sems_in[g])

  pltpu.sync_copy(inds_hbm.at[wid], inds_v)
  pltpu.sync_copy(w_hbm.at[wid], w_v)
  pltpu.sync_copy(ll_hbm.at[wid], ll_v)
  pltpu.sync_copy(pi_hbm.at[wid], pi_v)

  def gather_body(j, mx):
    sl = pl.ds(j * L, L)
    idx16 = inds_v[sl]
    wr16 = plsc.load_gather(w_v, [idx16])
    wr_v[sl] = wr16
    llr_v[sl] = plsc.load_gather(ll_v, [idx16])
    pir_v[sl] = plsc.load_gather(pi_v, [idx16])
    return jnp.maximum(mx, wr16)

  mx16 = lax.fori_loop(0, NVEC, gather_body,
                       jnp.full((L,), -jnp.inf, jnp.float32))
  m = lax.reduce_max_p.bind(mx16, axes=(0,))

  def exp_body(j, s):
    sl = pl.ds(j * L, L)
    e16 = jnp.exp(wr_v[sl] - m)
    wexp_v[sl] = e16
    return s + e16

  s16 = lax.fori_loop(0, NVEC, exp_body, jnp.zeros((L,), jnp.float32))
  z_vec = jnp.full((L,), lax.reduce_sum_p.bind(s16, axes=(0,)), jnp.float32)
  inv_z = jnp.full((L,), 1.0, jnp.float32) / z_vec

  pltpu.sync_copy(wr_v, wr_hbm.at[wid])
  pltpu.sync_copy(llr_v, llr_hbm.at[wid])
  pltpu.sync_copy(pir_v, pir_hbm.at[wid])

  lane0 = lax.iota(jnp.int32, L) == 0

  out_dma = {}
  for g in range(NG):
    ph = g % NBUF
    ibuf, obuf = ibufs[ph], obufs[ph]
    if g >= NBUF:
      out_dma[g - NBUF].wait()
    in_dma[g].wait()

    def group_body(j, accs, ibuf=ibuf, obuf=obuf):
      sl = pl.ds(j * L, L)
      idx16 = inds_v[sl]
      we16 = wexp_v[sl]
      new = []
      for r in range(RPG):
        g16 = plsc.load_gather(ibuf, [jnp.full((L,), r, jnp.int32), idx16])
        obuf[r, sl] = g16
        new.append(accs[r] + we16 * g16)
      return tuple(new)

    accs = lax.fori_loop(
        0, NVEC, group_body,
        tuple(jnp.zeros((L,), jnp.float32) for _ in range(RPG)))

    out_dma[g] = pltpu.async_copy(
        obuf, xrb.at[pl.ds(g * RPG, RPG)], sems_out[ph])
    if g + NBUF < NG:
      in_dma[g + NBUF] = pltpu.async_copy(
          xb.at[pl.ds((g + NBUF) * RPG, RPG)], ibuf, sems_in[ph])

    for r in range(RPG):
      s = lax.reduce_sum_p.bind(accs[r], axes=(0,))
      svec = jnp.full((L,), s, jnp.float32) * inv_z
      plsc.store_scatter(acc_v, [jnp.full((L,), g * RPG + r, jnp.int32)],
                         svec, mask=lane0)

  for g in range(NG - NBUF, NG):
    out_dma[g].wait()
  pltpu.sync_copy(acc_v, mean_hbm.at[wid])


@jax.jit
def kernel(x, w, ll, prev_inds, inds):
  xt = x.transpose(0, 2, 1)
  inds32 = inds.astype(jnp.int32)
  pi32 = prev_inds.astype(jnp.int32)

  mesh = plsc.VectorSubcoreMesh(core_axis_name="c", subcore_axis_name="s")
  run = pl.kernel(
      _sc_body,
      out_type=(
          jax.ShapeDtypeStruct((B, D), jnp.float32),
          jax.ShapeDtypeStruct((B, D, N), jnp.float32),
          jax.ShapeDtypeStruct((B, N), jnp.float32),
          jax.ShapeDtypeStruct((B, N), jnp.float32),
          jax.ShapeDtypeStruct((B, N), jnp.int32),
      ),
      mesh=mesh,
      compiler_params=pltpu.CompilerParams(needs_layout_passes=False,
                                           use_tc_tiling_on_sc=False),
      scratch_types=[
          pltpu.VMEM((N,), jnp.int32),
          pltpu.VMEM((N,), jnp.float32),
          pltpu.VMEM((N,), jnp.float32),
          pltpu.VMEM((N,), jnp.int32),
          pltpu.VMEM((N,), jnp.float32),
          pltpu.VMEM((N,), jnp.float32),
          pltpu.VMEM((N,), jnp.int32),
          pltpu.VMEM((N,), jnp.float32),
          pltpu.VMEM((D,), jnp.float32),
          pltpu.VMEM((RPG, N), jnp.float32),
          pltpu.VMEM((RPG, N), jnp.float32),
          pltpu.VMEM((RPG, N), jnp.float32),
          pltpu.VMEM((RPG, N), jnp.float32),
          pltpu.SemaphoreType.DMA,
          pltpu.SemaphoreType.DMA,
          pltpu.SemaphoreType.DMA,
          pltpu.SemaphoreType.DMA,
      ],
  )
  mean, xtr, wr, llr, pir = run(xt, w, ll, pi32, inds32)
  return (mean, xtr.transpose(0, 2, 1), wr, llr,
          pir.astype(prev_inds.dtype))
